# baseline (speedup 1.0000x reference)
import functools

import jax
import jax.numpy as jnp
from jax import lax
from jax.experimental import pallas as pl
from jax.experimental.pallas import tpu as pltpu
from jax.experimental.pallas import tpu_sc as plsc

N = 10000
E = 320000
NPAD = 10240
NCORES = 2
NSUB = 16
ROWS_PER_TILE = NPAD // NSUB
J = 4
R = 128
CHUNK = J * R
G = 20
EPAD = NCORES * NSUB * G * CHUNK
STAGE_R = 128


def _make_agg(D):
    mesh = plsc.VectorSubcoreMesh(core_axis_name="c", subcore_axis_name="s")

    @functools.partial(
        pl.kernel,
        out_type=jax.ShapeDtypeStruct((NCORES, NPAD, D), jnp.float32),
        mesh=mesh,
        scratch_types=[
            pltpu.VMEM((2, J, R), jnp.int32),
            pltpu.VMEM((2, J, R), jnp.int32),
            pltpu.VMEM((J, R, D), jnp.float32),
            pltpu.VMEM((STAGE_R, D), jnp.float32),
            pltpu.VMEM_SHARED((NPAD, D), jnp.float32),
            pltpu.VMEM_SHARED((NPAD, D), jnp.float32),
            pltpu.SemaphoreType.DMA,
            pltpu.SemaphoreType.DMA,
        ],
        compiler_params=pltpu.CompilerParams(use_tc_tiling_on_sc=False),
    )
    def agg(g_hbm, src_hbm, dst_hbm, zeros_hbm, out_hbm,
            sidx, didx, rows, stage, gspm, acc, gsem, isem):
        cid = lax.axis_index("c")
        sid = lax.axis_index("s")
        rlo = sid * ROWS_PER_TILE
        pltpu.sync_copy(zeros_hbm, stage)
        for k in range(ROWS_PER_TILE // STAGE_R):
            pltpu.sync_copy(stage, acc.at[pl.ds(rlo + k * STAGE_R, STAGE_R)])
        for k in range(ROWS_PER_TILE // STAGE_R):
            lo = rlo + k * STAGE_R
            pltpu.sync_copy(g_hbm.at[pl.ds(lo, STAGE_R)], stage)
            pltpu.sync_copy(stage, gspm.at[pl.ds(lo, STAGE_R)])
        plsc.subcore_barrier()

        def fire_idx(c, p):
            pltpu.async_copy(src_hbm.at[cid, sid, c], sidx.at[p], isem)
            pltpu.async_copy(dst_hbm.at[cid, sid, c], didx.at[p], isem)

        def wait_idx(c, p):
            pltpu.make_async_copy(src_hbm.at[cid, sid, c], sidx.at[p], isem).wait()
            pltpu.make_async_copy(dst_hbm.at[cid, sid, c], didx.at[p], isem).wait()

        pltpu.sync_copy(src_hbm.at[cid, sid, 0], sidx.at[0])
        pltpu.sync_copy(dst_hbm.at[cid, sid, 0], didx.at[0])

        def chunk_body(c, carry):
            p = lax.rem(c, 2)
            q = 1 - p

            @pl.when(c >= 1)
            def _wait_idx_prefetch():
                wait_idx(c, p)

            @pl.when(c + 1 < G)
            def _prefetch_idx():
                fire_idx(c + 1, q)

            for j in range(J):
                pltpu.async_copy(gspm.at[sidx.at[p, j]], rows.at[j], gsem)
            for j in range(J):
                pltpu.make_async_copy(
                    gspm.at[sidx.at[p, j]], rows.at[j], gsem).wait()
            for j in range(J):
                pltpu.sync_copy(rows.at[j], acc.at[didx.at[p, j]], add=True)
            return carry

        lax.fori_loop(0, G, chunk_body, 0)
        plsc.subcore_barrier()
        for k in range(ROWS_PER_TILE // STAGE_R):
            lo = rlo + k * STAGE_R
            pltpu.sync_copy(acc.at[pl.ds(lo, STAGE_R)], stage)
            pltpu.sync_copy(stage, out_hbm.at[cid, pl.ds(lo, STAGE_R)])

    return agg


_agg64 = _make_agg(64)
_agg32 = _make_agg(32)

_DEG_D = 16


def _make_deg():
    mesh = plsc.VectorSubcoreMesh(core_axis_name="c", subcore_axis_name="s")

    @functools.partial(
        pl.kernel,
        out_type=jax.ShapeDtypeStruct((NCORES, NPAD, _DEG_D), jnp.float32),
        mesh=mesh,
        scratch_types=[
            pltpu.VMEM((2, J, R), jnp.int32),
            pltpu.VMEM((J, R, _DEG_D), jnp.float32),
            pltpu.VMEM((STAGE_R, _DEG_D), jnp.float32),
            pltpu.VMEM_SHARED((NPAD, _DEG_D), jnp.float32),
            pltpu.SemaphoreType.DMA,
        ],
        compiler_params=pltpu.CompilerParams(use_tc_tiling_on_sc=False),
    )
    def deg(dst_hbm, zeros_hbm, ones_hbm, out_hbm, didx, ones_v, stage, acc, ssem):
        cid = lax.axis_index("c")
        sid = lax.axis_index("s")
        rlo = sid * ROWS_PER_TILE
        pltpu.sync_copy(zeros_hbm, stage)
        for k in range(ROWS_PER_TILE // STAGE_R):
            pltpu.sync_copy(stage, acc.at[pl.ds(rlo + k * STAGE_R, STAGE_R)])
        plsc.subcore_barrier()
        pltpu.sync_copy(ones_hbm, ones_v)
        pltpu.sync_copy(dst_hbm.at[cid, sid, 0], didx.at[0])

        def fire_scatters(p):
            for j in range(J):
                pltpu.async_copy(
                    ones_v.at[j], acc.at[didx.at[p, j]], ssem, add=True)

        def wait_scatters(p):
            for j in range(J):
                pltpu.make_async_copy(
                    ones_v.at[j], acc.at[didx.at[p, j]], ssem).wait()

        def chunk_body(c, carry):
            p = lax.rem(c, 2)
            q = 1 - p

            @pl.when(c >= 1)
            def _drain_prev_scatter():
                wait_scatters(q)

            @pl.when(c + 1 < G)
            def _prefetch():
                pltpu.sync_copy(dst_hbm.at[cid, sid, c + 1], didx.at[q])

            fire_scatters(p)
            return carry

        lax.fori_loop(0, G, chunk_body, 0)
        wait_scatters(1)
        plsc.subcore_barrier()
        for k in range(ROWS_PER_TILE // STAGE_R):
            lo = rlo + k * STAGE_R
            pltpu.sync_copy(acc.at[pl.ds(lo, STAGE_R)], stage)
            pltpu.sync_copy(stage, out_hbm.at[cid, pl.ds(lo, STAGE_R)])

    return deg


_deg_kernel = _make_deg()


def _dinv_from(deg_ref):
    d = deg_ref[0, :, 0:1] + deg_ref[1, :, 0:1] + 1.0
    return lax.rsqrt(jnp.maximum(d, 1e-12))


def _prep_body(deg_ref, x_ref, w_ref, o_ref):
    dinv = _dinv_from(deg_ref)
    h = jnp.dot(x_ref[...], w_ref[...], preferred_element_type=jnp.float32)
    o_ref[...] = h * dinv


def _combine_body(deg_ref, s_ref, g_ref, b_ref, w_ref, o_ref):
    dinv = _dinv_from(deg_ref)
    s = s_ref[0] + s_ref[1] + g_ref[...]
    xn = jnp.maximum(dinv * s + b_ref[...], 0.0)
    o_ref[...] = jnp.dot(xn, w_ref[...], preferred_element_type=jnp.float32) * dinv


def _final_body(deg_ref, s_ref, g_ref, b_ref, wp_ref, bp_ref, o_ref):
    dinv = _dinv_from(deg_ref)
    s = s_ref[0] + s_ref[1] + g_ref[...]
    xn = jnp.maximum(dinv * s + b_ref[...], 0.0)
    o_ref[...] = jnp.dot(xn, wp_ref[...], preferred_element_type=jnp.float32) + bp_ref[...]


def _tc_call(body, out_dim):
    return pl.pallas_call(body, out_shape=jax.ShapeDtypeStruct((NPAD, out_dim), jnp.float32))


def kernel(x, edge_index, W1, b1, W2, b2, W3, b3, Wp, bp):
    src = edge_index[0].astype(jnp.int32)
    dst = edge_index[1].astype(jnp.int32)
    pad = EPAD - E
    srcp = jnp.concatenate([src, jnp.zeros((pad,), jnp.int32)])
    dstp = jnp.concatenate([dst, jnp.full((pad,), N, jnp.int32)])
    src_r = srcp.reshape(NCORES, NSUB, G, J, R)
    dst_r = dstp.reshape(NCORES, NSUB, G, J, R)

    xp = jnp.zeros((NPAD, x.shape[1]), x.dtype).at[:N].set(x)

    zeros64 = jnp.zeros((STAGE_R, 64), jnp.float32)
    zeros32 = jnp.zeros((STAGE_R, 32), jnp.float32)
    zeros16 = jnp.zeros((STAGE_R, _DEG_D), jnp.float32)
    ones16 = jnp.ones((J, R, _DEG_D), jnp.float32)

    degP = _deg_kernel(dst_r, zeros16, ones16)

    g1 = _tc_call(_prep_body, 64)(degP, xp, W1)
    S1 = _agg64(g1, src_r, dst_r, zeros64)
    g2 = _tc_call(_combine_body, 64)(degP, S1, g1, b1.reshape(1, -1), W2)
    S2 = _agg64(g2, src_r, dst_r, zeros64)
    g3 = _tc_call(_combine_body, 32)(degP, S2, g2, b2.reshape(1, -1), W3)
    S3 = _agg32(g3, src_r, dst_r, zeros32)
    y = _tc_call(_final_body, 1)(degP, S3, g3, b3.reshape(1, -1), Wp, bp.reshape(1, 1))
    return y[:N]


# speedup vs baseline: 1.0148x; 1.0148x over previous
import functools

import jax
import jax.numpy as jnp
from jax import lax
from jax.experimental import pallas as pl
from jax.experimental.pallas import tpu as pltpu
from jax.experimental.pallas import tpu_sc as plsc

N = 10000
E = 320000
NPAD = 10240
NCORES = 2
NSUB = 16
ROWS_PER_TILE = NPAD // NSUB
J = 4
R = 128
CHUNK = J * R
G = 20
EPAD = NCORES * NSUB * G * CHUNK
STAGE_R = 128


def _make_agg(D):
    mesh = plsc.VectorSubcoreMesh(core_axis_name="c", subcore_axis_name="s")

    @functools.partial(
        pl.kernel,
        out_type=jax.ShapeDtypeStruct((NCORES, NPAD, D), jnp.float32),
        mesh=mesh,
        scratch_types=[
            pltpu.VMEM((2, CHUNK), jnp.int32),
            pltpu.VMEM((2, CHUNK), jnp.int32),
            pltpu.VMEM((CHUNK, D), jnp.float32),
            pltpu.VMEM((STAGE_R, D), jnp.float32),
            pltpu.VMEM_SHARED((NPAD, D), jnp.float32),
            pltpu.VMEM_SHARED((NPAD, D), jnp.float32),
            pltpu.SemaphoreType.DMA,
            pltpu.SemaphoreType.DMA,
        ],
        compiler_params=pltpu.CompilerParams(use_tc_tiling_on_sc=False),
    )
    def agg(g_hbm, src_hbm, dst_hbm, zeros_hbm, out_hbm,
            sidx, didx, rows, stage, gspm, acc, gsem, isem):
        cid = lax.axis_index("c")
        sid = lax.axis_index("s")
        rlo = sid * ROWS_PER_TILE
        pltpu.sync_copy(zeros_hbm, stage)
        for k in range(ROWS_PER_TILE // STAGE_R):
            pltpu.sync_copy(stage, acc.at[pl.ds(rlo + k * STAGE_R, STAGE_R)])
        for k in range(ROWS_PER_TILE // STAGE_R):
            lo = rlo + k * STAGE_R
            pltpu.sync_copy(g_hbm.at[pl.ds(lo, STAGE_R)], stage)
            pltpu.sync_copy(stage, gspm.at[pl.ds(lo, STAGE_R)])
        plsc.subcore_barrier()

        def fire_idx(c, p):
            pltpu.async_copy(src_hbm.at[cid, sid, c], sidx.at[p], isem)
            pltpu.async_copy(dst_hbm.at[cid, sid, c], didx.at[p], isem)

        def wait_idx(c, p):
            pltpu.make_async_copy(src_hbm.at[cid, sid, c], sidx.at[p], isem).wait()
            pltpu.make_async_copy(dst_hbm.at[cid, sid, c], didx.at[p], isem).wait()

        pltpu.sync_copy(src_hbm.at[cid, sid, 0], sidx.at[0])
        pltpu.sync_copy(dst_hbm.at[cid, sid, 0], didx.at[0])

        def chunk_body(c, carry):
            p = lax.rem(c, 2)
            q = 1 - p

            @pl.when(c >= 1)
            def _wait_idx_prefetch():
                wait_idx(c, p)

            @pl.when(c + 1 < G)
            def _prefetch_idx():
                fire_idx(c + 1, q)

            pltpu.sync_copy(gspm.at[sidx.at[p]], rows)
            pltpu.sync_copy(rows, acc.at[didx.at[p]], add=True)
            return carry

        lax.fori_loop(0, G, chunk_body, 0)
        plsc.subcore_barrier()
        for k in range(ROWS_PER_TILE // STAGE_R):
            lo = rlo + k * STAGE_R
            pltpu.sync_copy(acc.at[pl.ds(lo, STAGE_R)], stage)
            pltpu.sync_copy(stage, out_hbm.at[cid, pl.ds(lo, STAGE_R)])

    return agg


_agg64 = _make_agg(64)
_agg32 = _make_agg(32)

_DEG_D = 16


def _make_deg():
    mesh = plsc.VectorSubcoreMesh(core_axis_name="c", subcore_axis_name="s")

    @functools.partial(
        pl.kernel,
        out_type=jax.ShapeDtypeStruct((NCORES, NPAD, _DEG_D), jnp.float32),
        mesh=mesh,
        scratch_types=[
            pltpu.VMEM((2, CHUNK), jnp.int32),
            pltpu.VMEM((CHUNK, _DEG_D), jnp.float32),
            pltpu.VMEM((STAGE_R, _DEG_D), jnp.float32),
            pltpu.VMEM_SHARED((NPAD, _DEG_D), jnp.float32),
            pltpu.SemaphoreType.DMA,
        ],
        compiler_params=pltpu.CompilerParams(use_tc_tiling_on_sc=False),
    )
    def deg(dst_hbm, zeros_hbm, ones_hbm, out_hbm, didx, ones_v, stage, acc, ssem):
        cid = lax.axis_index("c")
        sid = lax.axis_index("s")
        rlo = sid * ROWS_PER_TILE
        pltpu.sync_copy(zeros_hbm, stage)
        for k in range(ROWS_PER_TILE // STAGE_R):
            pltpu.sync_copy(stage, acc.at[pl.ds(rlo + k * STAGE_R, STAGE_R)])
        plsc.subcore_barrier()
        pltpu.sync_copy(ones_hbm, ones_v)
        pltpu.sync_copy(dst_hbm.at[cid, sid, 0], didx.at[0])

        def fire_scatters(p):
            pltpu.async_copy(ones_v, acc.at[didx.at[p]], ssem, add=True)

        def wait_scatters(p):
            pltpu.make_async_copy(ones_v, acc.at[didx.at[p]], ssem).wait()

        def chunk_body(c, carry):
            p = lax.rem(c, 2)
            q = 1 - p

            @pl.when(c >= 1)
            def _drain_prev_scatter():
                wait_scatters(q)

            @pl.when(c + 1 < G)
            def _prefetch():
                pltpu.sync_copy(dst_hbm.at[cid, sid, c + 1], didx.at[q])

            fire_scatters(p)
            return carry

        lax.fori_loop(0, G, chunk_body, 0)
        wait_scatters(1)
        plsc.subcore_barrier()
        for k in range(ROWS_PER_TILE // STAGE_R):
            lo = rlo + k * STAGE_R
            pltpu.sync_copy(acc.at[pl.ds(lo, STAGE_R)], stage)
            pltpu.sync_copy(stage, out_hbm.at[cid, pl.ds(lo, STAGE_R)])

    return deg


_deg_kernel = _make_deg()


def _dinv_from(deg_ref):
    d = deg_ref[0, :, 0:1] + deg_ref[1, :, 0:1] + 1.0
    return lax.rsqrt(jnp.maximum(d, 1e-12))


def _prep_body(deg_ref, x_ref, w_ref, o_ref):
    dinv = _dinv_from(deg_ref)
    h = jnp.dot(x_ref[...], w_ref[...], preferred_element_type=jnp.float32)
    o_ref[...] = h * dinv


def _combine_body(deg_ref, s_ref, g_ref, b_ref, w_ref, o_ref):
    dinv = _dinv_from(deg_ref)
    s = s_ref[0] + s_ref[1] + g_ref[...]
    xn = jnp.maximum(dinv * s + b_ref[...], 0.0)
    o_ref[...] = jnp.dot(xn, w_ref[...], preferred_element_type=jnp.float32) * dinv


def _final_body(deg_ref, s_ref, g_ref, b_ref, wp_ref, bp_ref, o_ref):
    dinv = _dinv_from(deg_ref)
    s = s_ref[0] + s_ref[1] + g_ref[...]
    xn = jnp.maximum(dinv * s + b_ref[...], 0.0)
    o_ref[...] = jnp.dot(xn, wp_ref[...], preferred_element_type=jnp.float32) + bp_ref[...]


def _tc_call(body, out_dim):
    return pl.pallas_call(body, out_shape=jax.ShapeDtypeStruct((NPAD, out_dim), jnp.float32))


def kernel(x, edge_index, W1, b1, W2, b2, W3, b3, Wp, bp):
    src = edge_index[0].astype(jnp.int32)
    dst = edge_index[1].astype(jnp.int32)
    pad = EPAD - E
    srcp = jnp.concatenate([src, jnp.zeros((pad,), jnp.int32)])
    dstp = jnp.concatenate([dst, jnp.full((pad,), N, jnp.int32)])
    src_r = srcp.reshape(NCORES, NSUB, G, CHUNK)
    dst_r = dstp.reshape(NCORES, NSUB, G, CHUNK)

    xp = jnp.zeros((NPAD, x.shape[1]), x.dtype).at[:N].set(x)

    zeros64 = jnp.zeros((STAGE_R, 64), jnp.float32)
    zeros32 = jnp.zeros((STAGE_R, 32), jnp.float32)
    zeros16 = jnp.zeros((STAGE_R, _DEG_D), jnp.float32)
    ones16 = jnp.ones((CHUNK, _DEG_D), jnp.float32)

    degP = _deg_kernel(dst_r, zeros16, ones16)

    g1 = _tc_call(_prep_body, 64)(degP, xp, W1)
    S1 = _agg64(g1, src_r, dst_r, zeros64)
    g2 = _tc_call(_combine_body, 64)(degP, S1, g1, b1.reshape(1, -1), W2)
    S2 = _agg64(g2, src_r, dst_r, zeros64)
    g3 = _tc_call(_combine_body, 32)(degP, S2, g2, b2.reshape(1, -1), W3)
    S3 = _agg32(g3, src_r, dst_r, zeros32)
    y = _tc_call(_final_body, 1)(degP, S3, g3, b3.reshape(1, -1), Wp, bp.reshape(1, 1))
    return y[:N]


# speedup vs baseline: 1.1159x; 1.0996x over previous
import functools

import jax
import jax.numpy as jnp
from jax import lax
from jax.experimental import pallas as pl
from jax.experimental.pallas import tpu as pltpu
from jax.experimental.pallas import tpu_sc as plsc

N = 10000
E = 320000
NPAD = 10240
NCORES = 2
NSUB = 16
ROWS_PER_TILE = NPAD // NSUB
CHUNK = 256
G = 40
EPAD = NCORES * NSUB * G * CHUNK
STAGE_R = 128


def _make_agg(D):
    mesh = plsc.VectorSubcoreMesh(core_axis_name="c", subcore_axis_name="s")

    @functools.partial(
        pl.kernel,
        out_type=jax.ShapeDtypeStruct((NCORES, NPAD, D), jnp.float32),
        mesh=mesh,
        scratch_types=[
            pltpu.VMEM((3, CHUNK), jnp.int32),
            pltpu.VMEM((3, CHUNK), jnp.int32),
            pltpu.VMEM((2, CHUNK, D), jnp.float32),
            pltpu.VMEM((STAGE_R, D), jnp.float32),
            pltpu.VMEM_SHARED((NPAD, D), jnp.float32),
            pltpu.VMEM_SHARED((NPAD, D), jnp.float32),
            pltpu.SemaphoreType.DMA,
            pltpu.SemaphoreType.DMA,
        ],
        compiler_params=pltpu.CompilerParams(use_tc_tiling_on_sc=False),
    )
    def agg(g_hbm, src_hbm, dst_hbm, zeros_hbm, out_hbm,
            sidx, didx, rows, stage, gspm, acc, gsem, isem):
        cid = lax.axis_index("c")
        sid = lax.axis_index("s")
        rlo = sid * ROWS_PER_TILE
        pltpu.sync_copy(zeros_hbm, stage)
        for k in range(ROWS_PER_TILE // STAGE_R):
            pltpu.sync_copy(stage, acc.at[pl.ds(rlo + k * STAGE_R, STAGE_R)])
        for k in range(ROWS_PER_TILE // STAGE_R):
            lo = rlo + k * STAGE_R
            pltpu.sync_copy(g_hbm.at[pl.ds(lo, STAGE_R)], stage)
            pltpu.sync_copy(stage, gspm.at[pl.ds(lo, STAGE_R)])
        plsc.subcore_barrier()

        def fire_idx(c):
            k = lax.rem(c, 3)
            pltpu.async_copy(src_hbm.at[cid, sid, c], sidx.at[k], isem)
            pltpu.async_copy(dst_hbm.at[cid, sid, c], didx.at[k], isem)

        def wait_idx(c):
            k = lax.rem(c, 3)
            pltpu.make_async_copy(src_hbm.at[cid, sid, c], sidx.at[k], isem).wait()
            pltpu.make_async_copy(dst_hbm.at[cid, sid, c], didx.at[k], isem).wait()

        def fire_gather(c, p):
            pltpu.async_copy(gspm.at[sidx.at[lax.rem(c, 3)]], rows.at[p], gsem)

        def wait_gather(c, p):
            pltpu.make_async_copy(
                gspm.at[sidx.at[lax.rem(c, 3)]], rows.at[p], gsem).wait()

        pltpu.sync_copy(src_hbm.at[cid, sid, 0], sidx.at[0])
        pltpu.sync_copy(dst_hbm.at[cid, sid, 0], didx.at[0])
        fire_idx(1)
        fire_gather(0, 0)

        def chunk_body(c, carry):
            p = lax.rem(c, 2)
            q = 1 - p

            @pl.when(c + 1 < G)
            def _next_gather():
                wait_idx(c + 1)
                @pl.when(c + 2 < G)
                def _prefetch_idx():
                    fire_idx(c + 2)
                fire_gather(c + 1, q)

            wait_gather(c, p)
            pltpu.sync_copy(rows.at[p], acc.at[didx.at[lax.rem(c, 3)]], add=True)
            return carry

        lax.fori_loop(0, G, chunk_body, 0)
        plsc.subcore_barrier()
        for k in range(ROWS_PER_TILE // STAGE_R):
            lo = rlo + k * STAGE_R
            pltpu.sync_copy(acc.at[pl.ds(lo, STAGE_R)], stage)
            pltpu.sync_copy(stage, out_hbm.at[cid, pl.ds(lo, STAGE_R)])

    return agg


_agg64 = _make_agg(64)
_agg32 = _make_agg(32)

_DEG_D = 16


def _make_deg():
    mesh = plsc.VectorSubcoreMesh(core_axis_name="c", subcore_axis_name="s")

    @functools.partial(
        pl.kernel,
        out_type=jax.ShapeDtypeStruct((NCORES, NPAD, _DEG_D), jnp.float32),
        mesh=mesh,
        scratch_types=[
            pltpu.VMEM((2, CHUNK), jnp.int32),
            pltpu.VMEM((CHUNK, _DEG_D), jnp.float32),
            pltpu.VMEM((STAGE_R, _DEG_D), jnp.float32),
            pltpu.VMEM_SHARED((NPAD, _DEG_D), jnp.float32),
            pltpu.SemaphoreType.DMA,
        ],
        compiler_params=pltpu.CompilerParams(use_tc_tiling_on_sc=False),
    )
    def deg(dst_hbm, zeros_hbm, ones_hbm, out_hbm, didx, ones_v, stage, acc, ssem):
        cid = lax.axis_index("c")
        sid = lax.axis_index("s")
        rlo = sid * ROWS_PER_TILE
        pltpu.sync_copy(zeros_hbm, stage)
        for k in range(ROWS_PER_TILE // STAGE_R):
            pltpu.sync_copy(stage, acc.at[pl.ds(rlo + k * STAGE_R, STAGE_R)])
        plsc.subcore_barrier()
        pltpu.sync_copy(ones_hbm, ones_v)
        pltpu.sync_copy(dst_hbm.at[cid, sid, 0], didx.at[0])

        def fire_scatters(p):
            pltpu.async_copy(ones_v, acc.at[didx.at[p]], ssem, add=True)

        def wait_scatters(p):
            pltpu.make_async_copy(ones_v, acc.at[didx.at[p]], ssem).wait()

        def chunk_body(c, carry):
            p = lax.rem(c, 2)
            q = 1 - p

            @pl.when(c >= 1)
            def _drain_prev_scatter():
                wait_scatters(q)

            @pl.when(c + 1 < G)
            def _prefetch():
                pltpu.sync_copy(dst_hbm.at[cid, sid, c + 1], didx.at[q])

            fire_scatters(p)
            return carry

        lax.fori_loop(0, G, chunk_body, 0)
        wait_scatters(1)
        plsc.subcore_barrier()
        for k in range(ROWS_PER_TILE // STAGE_R):
            lo = rlo + k * STAGE_R
            pltpu.sync_copy(acc.at[pl.ds(lo, STAGE_R)], stage)
            pltpu.sync_copy(stage, out_hbm.at[cid, pl.ds(lo, STAGE_R)])

    return deg


_deg_kernel = _make_deg()


def _dinv_from(deg_ref):
    d = deg_ref[0, :, 0:1] + deg_ref[1, :, 0:1] + 1.0
    return lax.rsqrt(jnp.maximum(d, 1e-12))


def _prep_body(deg_ref, x_ref, w_ref, o_ref):
    dinv = _dinv_from(deg_ref)
    h = jnp.dot(x_ref[...], w_ref[...], preferred_element_type=jnp.float32)
    o_ref[...] = h * dinv


def _combine_body(deg_ref, s_ref, g_ref, b_ref, w_ref, o_ref):
    dinv = _dinv_from(deg_ref)
    s = s_ref[0] + s_ref[1] + g_ref[...]
    xn = jnp.maximum(dinv * s + b_ref[...], 0.0)
    o_ref[...] = jnp.dot(xn, w_ref[...], preferred_element_type=jnp.float32) * dinv


def _final_body(deg_ref, s_ref, g_ref, b_ref, wp_ref, bp_ref, o_ref):
    dinv = _dinv_from(deg_ref)
    s = s_ref[0] + s_ref[1] + g_ref[...]
    xn = jnp.maximum(dinv * s + b_ref[...], 0.0)
    o_ref[...] = jnp.dot(xn, wp_ref[...], preferred_element_type=jnp.float32) + bp_ref[...]


def _tc_call(body, out_dim):
    return pl.pallas_call(body, out_shape=jax.ShapeDtypeStruct((NPAD, out_dim), jnp.float32))


def kernel(x, edge_index, W1, b1, W2, b2, W3, b3, Wp, bp):
    src = edge_index[0].astype(jnp.int32)
    dst = edge_index[1].astype(jnp.int32)
    pad = EPAD - E
    srcp = jnp.concatenate([src, jnp.zeros((pad,), jnp.int32)])
    dstp = jnp.concatenate([dst, jnp.full((pad,), N, jnp.int32)])
    src_r = srcp.reshape(NCORES, NSUB, G, CHUNK)
    dst_r = dstp.reshape(NCORES, NSUB, G, CHUNK)

    xp = jnp.zeros((NPAD, x.shape[1]), x.dtype).at[:N].set(x)

    zeros64 = jnp.zeros((STAGE_R, 64), jnp.float32)
    zeros32 = jnp.zeros((STAGE_R, 32), jnp.float32)
    zeros16 = jnp.zeros((STAGE_R, _DEG_D), jnp.float32)
    ones16 = jnp.ones((CHUNK, _DEG_D), jnp.float32)

    degP = _deg_kernel(dst_r, zeros16, ones16)

    g1 = _tc_call(_prep_body, 64)(degP, xp, W1)
    S1 = _agg64(g1, src_r, dst_r, zeros64)
    g2 = _tc_call(_combine_body, 64)(degP, S1, g1, b1.reshape(1, -1), W2)
    S2 = _agg64(g2, src_r, dst_r, zeros64)
    g3 = _tc_call(_combine_body, 32)(degP, S2, g2, b2.reshape(1, -1), W3)
    S3 = _agg32(g3, src_r, dst_r, zeros32)
    y = _tc_call(_final_body, 1)(degP, S3, g3, b3.reshape(1, -1), Wp, bp.reshape(1, 1))
    return y[:N]
